# baseline (speedup 1.0000x reference)
import functools
import jax
import jax.numpy as jnp
from jax import lax
from jax.experimental import pallas as pl
from jax.experimental.pallas import tpu as pltpu
from jax.experimental.pallas import tpu_sc as plsc

A, B, N = 64, 32, 8192
ROWS = A * B
NW = 32
ROWS_PER_W = ROWS // NW
LANES = 16
NV = N // LANES
RANK = N - 1 - N // 2

MINI = -(2 ** 31)
MASK31 = 0x7FFFFFFF
CAND = N + LANES


def _splat(v, dtype=jnp.int32):
  return lax.broadcast(jnp.asarray(v, dtype), (LANES,))


@functools.partial(
    pl.kernel,
    out_type=jax.ShapeDtypeStruct((ROWS,), jnp.int32),
    mesh=plsc.VectorSubcoreMesh(core_axis_name="c", subcore_axis_name="s"),
    compiler_params=pltpu.CompilerParams(needs_layout_passes=False),
    scratch_types=[
        pltpu.VMEM((N,), jnp.int32),
        pltpu.VMEM((CAND,), jnp.int32),
        pltpu.VMEM((CAND,), jnp.int32),
        pltpu.VMEM((256,), jnp.int32),
        pltpu.VMEM((ROWS_PER_W,), jnp.int32),
    ],
)
def _select_kernel(x_hbm, out_hbm, raw_v, ca_v, cb_v, h_v, res_v):
  cid = lax.axis_index("c")
  sid = lax.axis_index("s")
  wid = sid * 2 + cid
  base_row = wid * ROWS_PER_W
  lane = lax.broadcasted_iota(jnp.int32, (LANES,), 0)
  zero = _splat(0)
  one = _splat(1)
  maxi = _splat(MASK31)

  def zero_hist():
    for g in range(16):
      h_v[pl.ds(g * LANES, LANES)] = zero

  def locate(r_spl):
    def g_body(g, carry):
      acc_b, acc_rb, run = carry
      hv = h_v[pl.ds(g * LANES, LANES)]
      cs = plsc.cumsum(hv)
      below = run + cs - hv
      hit = (below <= r_spl) & (below + hv > r_spl)
      acc_b = acc_b + jnp.where(hit, lax.broadcast(g * LANES, (LANES,)) + lane,
                                zero)
      acc_rb = acc_rb + jnp.where(hit, below, zero)
      run = run + lax.broadcast(jnp.sum(hv), (LANES,))
      return acc_b, acc_rb, run
    acc_b, acc_rb, _ = lax.fori_loop(0, 16, g_body, (zero, zero, zero))
    b = lax.broadcast(jnp.max(acc_b), (LANES,))
    rb = lax.broadcast(jnp.max(acc_rb), (LANES,))
    return b, rb

  def key_of_raw(j):
    i = raw_v[pl.ds(j * LANES, LANES)]
    return jnp.where(i < 0, i ^ MASK31, i)

  def per_row(r, carry):
    pltpu.sync_copy(x_hbm.at[base_row + r], raw_v)

    zero_hist()

    def l1(j, _):
      ub = key_of_raw(j) ^ MINI
      d = lax.shift_right_logical(ub, 24)
      plsc.addupdate_scatter(h_v, [d], one)
      return 0

    lax.fori_loop(0, NV, l1, 0, unroll=8)
    b1, rb = locate(_splat(RANK))
    r_spl = _splat(RANK) - rb

    def c1(j, base):
      k = key_of_raw(j)
      d = lax.shift_right_logical(k ^ MINI, 24)
      m = d == b1
      mi = jnp.where(m, one, zero)
      idx = jnp.maximum(base + plsc.cumsum(mi) - 1, zero)
      plsc.store_scatter(ca_v, [idx], k, mask=m)
      return base + plsc.all_reduce_population_count(m)

    n_spl = lax.fori_loop(0, NV, c1, zero, unroll=8)
    plsc.store_scatter(ca_v, [n_spl + lane], maxi)

    def level(src_v, dst_v, shift, n_spl, r_spl):
      nv = lax.shift_right_logical(jnp.max(n_spl) + (LANES - 1), 4)
      zero_hist()

      def lh(j, _):
        ub = src_v[pl.ds(j * LANES, LANES)] ^ MINI
        d = lax.shift_right_logical(ub, shift) & 255
        plsc.addupdate_scatter(h_v, [d], one)
        return 0

      lax.fori_loop(0, nv, lh, 0)
      b_l, rb_l = locate(r_spl)
      r_out = r_spl - rb_l

      def lc(j, base):
        k = src_v[pl.ds(j * LANES, LANES)]
        d = lax.shift_right_logical(k ^ MINI, shift) & 255
        m = d == b_l
        mi = jnp.where(m, one, zero)
        idx = jnp.maximum(base + plsc.cumsum(mi) - 1, zero)
        plsc.store_scatter(dst_v, [idx], k, mask=m)
        return base + plsc.all_reduce_population_count(m)

      if shift > 0:
        n_out = lax.fori_loop(0, nv, lc, zero)
        plsc.store_scatter(dst_v, [n_out + lane], maxi)
      else:
        n_out = n_spl
      return b_l, n_out, r_out

    b2, n_spl, r_spl = level(ca_v, cb_v, 16, n_spl, r_spl)
    b3, n_spl, r_spl = level(cb_v, ca_v, 8, n_spl, r_spl)
    b4, _, _ = level(ca_v, cb_v, 0, n_spl, r_spl)

    ub_ans = (lax.shift_left(b1, 24) | lax.shift_left(b2, 16)
              | lax.shift_left(b3, 8) | b4)
    k_ans = ub_ans ^ MINI
    i_ans = jnp.where(k_ans < 0, k_ans ^ MASK31, k_ans)
    plsc.store_scatter(res_v, [lax.broadcast(r, (LANES,))], i_ans,
                       mask=lane == 0)
    return carry

  lax.fori_loop(0, ROWS_PER_W, per_row, 0)
  pltpu.sync_copy(res_v, out_hbm.at[pl.ds(base_row, ROWS_PER_W)])


def kernel(x):
  bits = lax.bitcast_convert_type(x.reshape(ROWS, N), jnp.int32)
  out = _select_kernel(bits)
  return lax.bitcast_convert_type(out, jnp.float32).reshape(A, B)
